# baseline (speedup 1.0000x reference)
import jax
import jax.numpy as jnp
from jax import lax
from jax.experimental import pallas as pl


def _logits_body(logits_ref, preds_ref, conf_ref):
    n = logits_ref.shape[1]
    L = logits_ref[0]
    m = jnp.max(L, axis=0)
    ii = lax.broadcasted_iota(jnp.int32, (n, n), 0)
    t = L - m[None, :]
    preds_ref[0, 0] = jnp.min(jnp.where(t == 0.0, ii, n), axis=0)
    conf_ref[0, 0] = 1.0 / jnp.sum(jnp.exp(t), axis=0)


def _recon_body(preds_ref, feat_ref, out_ref):
    n = feat_ref.shape[1]
    preds = preds_ref[0, 0]
    ii = lax.broadcasted_iota(jnp.int32, (n, n), 0)
    pp = lax.broadcasted_iota(jnp.int32, (n, n), 1)
    winner = jnp.max(jnp.where(preds[:, None] == pp, ii, -1), axis=0)
    jj16 = lax.broadcasted_iota(jnp.int16, (n, n), 0)
    one = jnp.bfloat16(1.0)
    zero = jnp.bfloat16(0.0)
    Pt = jnp.where(jj16 == winner[None, :].astype(jnp.int16), one, zero)

    inner = (Pt[:, :-2] + Pt[:, 1:-1] + Pt[:, 2:]) * jnp.bfloat16(1.0 / 3.0)
    Mt = jnp.concatenate([Pt[:, :1], inner, Pt[:, -1:]], axis=1)

    out_ref[0] = lax.dot_general(
        feat_ref[0].astype(jnp.bfloat16), Mt,
        dimension_numbers=(((0,), (0,)), ((), ())),
        preferred_element_type=jnp.float32,
    )


def kernel(features, position_logits):
    b, n, d = features.shape
    preds3, conf3 = pl.pallas_call(
        _logits_body,
        grid=(b,),
        in_specs=[pl.BlockSpec((1, n, n), lambda i: (i, 0, 0))],
        out_specs=[
            pl.BlockSpec((1, 1, n), lambda i: (i, 0, 0)),
            pl.BlockSpec((1, 1, n), lambda i: (i, 0, 0)),
        ],
        out_shape=[
            jax.ShapeDtypeStruct((b, 1, n), jnp.int32),
            jax.ShapeDtypeStruct((b, 1, n), jnp.float32),
        ],
    )(position_logits)
    recon_t = pl.pallas_call(
        _recon_body,
        grid=(b,),
        in_specs=[
            pl.BlockSpec((1, 1, n), lambda i: (i, 0, 0)),
            pl.BlockSpec((1, n, d), lambda i: (i, 0, 0)),
        ],
        out_specs=pl.BlockSpec((1, d, n), lambda i: (i, 0, 0)),
        out_shape=jax.ShapeDtypeStruct((b, d, n), jnp.float32),
    )(preds3, features)
    g = int(round(n ** 0.5))
    return (recon_t.reshape(b, d, g, g), conf3.reshape(b, n))
